# baseline (speedup 1.0000x reference)
import functools

import jax
import jax.numpy as jnp
from jax import lax
from jax.experimental import pallas as pl
from jax.experimental.pallas import tpu as pltpu
from jax.experimental.pallas import tpu_sc as plsc

B, N, G, C = 8, 4096, 64, 64
LANES = 16
SUBCORES = 32
PTS_PER_W = (B * N) // SUBCORES
VECS_PER_W = PTS_PER_W // LANES
QUARTERS = SUBCORES // B
BKT = C * G
WD = 1.0
WA = 1.0


def _rsqrt_nr(x):
    i = lax.bitcast_convert_type(x, jnp.int32)
    i = jnp.int32(0x5F3759DF) - (i >> 1)
    y = lax.bitcast_convert_type(i, jnp.float32)
    for _ in range(2):
        y = y * (1.5 - 0.5 * x * y * y)
    return y


def _sc_body(pts_hbm, off_hbm, lbl_hbm, ctr_hbm, plab_hbm, pres_hbm,
             pd_hbm, pa_hbm,
             praw, oraw, lblv, ctrraw, plabv, presv, cntv,
             bcx, bcy, bcz, bpres, accd, acca, sem):
    wid = lax.axis_index("s") * 2 + lax.axis_index("c")
    b = wid // QUARTERS
    n0 = (wid % QUARTERS) * PTS_PER_W

    copies = [
        pltpu.async_copy(pts_hbm.at[pl.ds(3 * (b * N + n0), 3 * PTS_PER_W)],
                         praw, sem),
        pltpu.async_copy(off_hbm.at[pl.ds(3 * (b * N + n0), 3 * PTS_PER_W)],
                         oraw, sem),
        pltpu.async_copy(lbl_hbm.at[pl.ds(b * N + n0, PTS_PER_W)], lblv, sem),
        pltpu.async_copy(ctr_hbm.at[pl.ds(b * 3 * G, 3 * G)], ctrraw, sem),
        pltpu.async_copy(plab_hbm.at[pl.ds(b * G, G)], plabv, sem),
        pltpu.async_copy(pres_hbm.at[pl.ds(b * G, G)], presv, sem),
    ]

    zf16 = jnp.zeros((LANES,), jnp.float32)

    def zinit(i, _):
        for t in range(4):
            s = pl.ds((i * 4 + t) * LANES, LANES)
            bcx[s] = zf16
            bcy[s] = zf16
            bcz[s] = zf16
            bpres[s] = zf16
        return 0

    lax.fori_loop(0, BKT // (4 * LANES), zinit, 0)
    zero16 = jnp.zeros((LANES,), jnp.int32)
    for i in range(C // LANES):
        cntv[pl.ds(i * LANES, LANES)] = zero16

    for cp in copies:
        cp.wait()

    lane = lax.iota(jnp.int32, LANES)
    lane0 = lane == 0

    def bucket(g, _):
        gsplat = jnp.full((LANES,), g, jnp.int32)
        lg = plsc.load_gather(plabv, [gsplat])
        kg = plsc.load_gather(cntv, [lg])
        slot = lg * G + kg
        g3 = jnp.full((LANES,), 3 * g, jnp.int32)
        cxg = plsc.load_gather(ctrraw, [g3])
        cyg = plsc.load_gather(ctrraw, [g3 + 1])
        czg = plsc.load_gather(ctrraw, [g3 + 2])
        pg = plsc.load_gather(presv, [gsplat])
        plsc.store_scatter(bcx, [slot], cxg, mask=lane0)
        plsc.store_scatter(bcy, [slot], cyg, mask=lane0)
        plsc.store_scatter(bcz, [slot], czg, mask=lane0)
        plsc.store_scatter(bpres, [slot], pg, mask=lane0)
        plsc.store_scatter(cntv, [lg], kg + 1, mask=lane0)
        return 0

    lax.fori_loop(0, G, bucket, 0)

    ILEAVE = 4
    lane3 = lane * 3

    def point_group(t, carry):
        ad, aa = carry
        vecs = []
        kmax_v = None
        for j in range(ILEAVE):
            s = pl.ds((t * ILEAVE + j) * LANES, LANES)
            pidx = jnp.full((LANES,), (t * ILEAVE + j) * (3 * LANES),
                            jnp.int32) + lane3
            px = plsc.load_gather(praw, [pidx])
            py = plsc.load_gather(praw, [pidx + 1])
            pz = plsc.load_gather(praw, [pidx + 2])
            ox = plsc.load_gather(oraw, [pidx])
            oy = plsc.load_gather(oraw, [pidx + 1])
            oz = plsc.load_gather(oraw, [pidx + 2])
            lbl = lblv[s]
            o2 = ox * ox + oy * oy + oz * oz
            kcount = plsc.load_gather(cntv, [lbl])
            vecs.append((px, py, pz, ox, oy, oz, lbl * G, o2))
            kmax_v = kcount if j == 0 else jnp.maximum(kmax_v, kcount)
        kmax = jnp.max(kmax_v)

        def pair(k, carry2):
            ad2, aa2 = carry2
            kk = jnp.full((LANES,), k, jnp.int32)
            for px, py, pz, ox, oy, oz, lblg, o2 in vecs:
                slot = lblg + kk
                cx = plsc.load_gather(bcx, [slot])
                cy = plsc.load_gather(bcy, [slot])
                cz = plsc.load_gather(bcz, [slot])
                w = plsc.load_gather(bpres, [slot])
                cvx, cvy, cvz = cx - px, cy - py, cz - pz
                dist = (jnp.abs(ox - cvx) + jnp.abs(oy - cvy)
                        + jnp.abs(oz - cvz))
                n2 = cvx * cvx + cvy * cvy + cvz * cvz
                dot = ox * cvx + oy * cvy + oz * cvz
                ang = -dot * _rsqrt_nr(o2 * n2)
                ad2 = ad2 + dist * w
                aa2 = aa2 + ang * w
            return ad2, aa2

        return lax.fori_loop(0, kmax, pair, (ad, aa))

    zf = jnp.zeros((LANES,), jnp.float32)
    ad, aa = lax.fori_loop(0, VECS_PER_W // ILEAVE, point_group, (zf, zf))
    accd[...] = ad
    acca[...] = aa
    pltpu.sync_copy(accd, pd_hbm.at[pl.ds(wid * LANES, LANES)])
    pltpu.sync_copy(acca, pa_hbm.at[pl.ds(wid * LANES, LANES)])


_sc_kernel = functools.partial(
    pl.kernel,
    out_type=[
        jax.ShapeDtypeStruct((SUBCORES * LANES,), jnp.float32),
        jax.ShapeDtypeStruct((SUBCORES * LANES,), jnp.float32),
    ],
    mesh=plsc.VectorSubcoreMesh(core_axis_name="c", subcore_axis_name="s"),
    compiler_params=pltpu.CompilerParams(needs_layout_passes=False),
    scratch_types=[
        pltpu.VMEM((3 * PTS_PER_W,), jnp.float32),
        pltpu.VMEM((3 * PTS_PER_W,), jnp.float32),
        pltpu.VMEM((PTS_PER_W,), jnp.int32),
        pltpu.VMEM((3 * G,), jnp.float32),
        pltpu.VMEM((G,), jnp.int32),
        pltpu.VMEM((G,), jnp.float32),
        pltpu.VMEM((C,), jnp.int32),
        pltpu.VMEM((BKT,), jnp.float32),
        pltpu.VMEM((BKT,), jnp.float32),
        pltpu.VMEM((BKT,), jnp.float32),
        pltpu.VMEM((BKT,), jnp.float32),
        pltpu.VMEM((LANES,), jnp.float32),
        pltpu.VMEM((LANES,), jnp.float32),
        pltpu.SemaphoreType.DMA,
    ],
)(_sc_body)


def _combine_body(pd_ref, pa_ref, of_ref, od_ref, oa_ref):
    scale = jnp.float32(1.0 / (N * B))
    sd = jnp.sum(pd_ref[...]) * (WD * scale)
    sa = jnp.sum(pa_ref[...]) * (WA * scale)
    of_ref[...] = jnp.full((1, 1), sd + sa, jnp.float32)
    od_ref[...] = jnp.full((1, 1), sd, jnp.float32)
    oa_ref[...] = jnp.full((1, 1), sa, jnp.float32)


_combine = pl.pallas_call(
    _combine_body,
    out_shape=[
        jax.ShapeDtypeStruct((1, 1), jnp.float32),
        jax.ShapeDtypeStruct((1, 1), jnp.float32),
        jax.ShapeDtypeStruct((1, 1), jnp.float32),
    ],
)


def kernel(point_normalized, point_xyz, point_offset, point_label,
           gt_plane_center, gt_plane_center_normalized,
           gt_center_sem_cls_label, gt_plane_present):
    pts = point_xyz.reshape(-1)
    off = point_offset.reshape(-1)
    ctr = gt_plane_center_normalized.reshape(-1)
    lbl = point_label.astype(jnp.int32).reshape(-1)
    plab = gt_center_sem_cls_label.astype(jnp.int32).reshape(-1)
    pres = gt_plane_present.astype(jnp.float32).reshape(-1)
    pd, pa = _sc_kernel(pts, off, lbl, ctr, plab, pres)
    f, d, a = _combine(pd, pa)
    return f[0, 0], d[0, 0], a[0, 0]
